# Initial kernel scaffold; baseline (speedup 1.0000x reference)
import jax
import jax.numpy as jnp
from jax.experimental import pallas as pl


def kernel(node, edge_index, edge, hidden, W1, g1, b1, W2, g2, b2, W3, g3, b3, W4, g4, b4, W_ih, W_hh, b_ih, b_hh, bias):
    raise NotImplementedError("write your pallas kernel here")


# speedup vs baseline: 1.0288x; 1.0288x over previous
import jax
import jax.numpy as jnp
from jax.experimental import pallas as pl
from jax.experimental.pallas import tpu as pltpu
from jax.experimental.pallas import tpu_sc as plsc

_N = 50000
_E = 100000
_EP = 102400
_D = 32
_EB = 2000
_NB = 2000
_CW = 128
_NH = 25088
_NAH = 25600
_SRH = 1600
_NCH2 = 50
_NBH = 1568
_EPS = 1e-5


def _sc_mesh():
    return plsc.VectorSubcoreMesh(core_axis_name="core", subcore_axis_name="subcore")


def _lin_first_body(x_ref, w_ref, y_ref, mom_ref):
    i = pl.program_id(0)
    y = jnp.dot(x_ref[...], w_ref[...].T, preferred_element_type=jnp.float32)
    y_ref[...] = y

    @pl.when(i == 0)
    def _():
        mom_ref[...] = jnp.zeros_like(mom_ref)

    mom_ref[0:1, :] += jnp.sum(y, axis=0, keepdims=True)
    mom_ref[1:2, :] += jnp.sum(y * y, axis=0, keepdims=True)


def _lin_first(x, W):
    e, cin = x.shape
    cout = W.shape[0]
    return pl.pallas_call(
        _lin_first_body,
        grid=(e // _EB,),
        in_specs=[
            pl.BlockSpec((_EB, cin), lambda i: (i, 0)),
            pl.BlockSpec((cout, cin), lambda i: (0, 0)),
        ],
        out_specs=[
            pl.BlockSpec((_EB, cout), lambda i: (i, 0)),
            pl.BlockSpec((8, cout), lambda i: (0, 0)),
        ],
        out_shape=[
            jax.ShapeDtypeStruct((e, cout), jnp.float32),
            jax.ShapeDtypeStruct((8, cout), jnp.float32),
        ],
    )(x, W)


def _bn_relu(y_ref, mom_ref, g_ref, b_ref):
    inv_e = 1.0 / _E
    mean = mom_ref[0:1, :] * inv_e
    var = mom_ref[1:2, :] * inv_e - mean * mean
    a = g_ref[...] * jax.lax.rsqrt(var + _EPS)
    c = b_ref[...] - mean * a
    return jnp.maximum(y_ref[...] * a + c, 0.0)


def _lin_mid_body(y_ref, mom_ref, g_ref, b_ref, w_ref, yo_ref, mo_ref):
    i = pl.program_id(0)
    h = _bn_relu(y_ref, mom_ref, g_ref, b_ref)
    y = jnp.dot(h, w_ref[...].T, preferred_element_type=jnp.float32)
    yo_ref[...] = y

    @pl.when(i == 0)
    def _():
        mo_ref[...] = jnp.zeros_like(mo_ref)

    mo_ref[0:1, :] += jnp.sum(y, axis=0, keepdims=True)
    mo_ref[1:2, :] += jnp.sum(y * y, axis=0, keepdims=True)


def _lin_mid(y, mom, g, b, W):
    e, cin = y.shape
    cout = W.shape[0]
    return pl.pallas_call(
        _lin_mid_body,
        grid=(e // _EB,),
        in_specs=[
            pl.BlockSpec((_EB, cin), lambda i: (i, 0)),
            pl.BlockSpec((8, cin), lambda i: (0, 0)),
            pl.BlockSpec((1, cin), lambda i: (0, 0)),
            pl.BlockSpec((1, cin), lambda i: (0, 0)),
            pl.BlockSpec((cout, cin), lambda i: (0, 0)),
        ],
        out_specs=[
            pl.BlockSpec((_EB, cout), lambda i: (i, 0)),
            pl.BlockSpec((8, cout), lambda i: (0, 0)),
        ],
        out_shape=[
            jax.ShapeDtypeStruct((e, cout), jnp.float32),
            jax.ShapeDtypeStruct((8, cout), jnp.float32),
        ],
    )(y, mom, g.reshape(1, -1), b.reshape(1, -1), W)


def _gram_body(y_ref, mom_ref, g_ref, b_ref, gram_ref, hs_ref):
    i = pl.program_id(0)
    h = _bn_relu(y_ref, mom_ref, g_ref, b_ref)

    @pl.when(i == 0)
    def _():
        gram_ref[...] = jnp.zeros_like(gram_ref)
        hs_ref[...] = jnp.zeros_like(hs_ref)

    gram_ref[...] += jax.lax.dot_general(
        h, h, (((0,), (0,)), ((), ())), preferred_element_type=jnp.float32)
    hs_ref[0:1, :] += jnp.sum(h, axis=0, keepdims=True)


def _gram(y, mom, g, b):
    e, c = y.shape
    return pl.pallas_call(
        _gram_body,
        grid=(e // _EB,),
        in_specs=[
            pl.BlockSpec((_EB, c), lambda i: (i, 0)),
            pl.BlockSpec((8, c), lambda i: (0, 0)),
            pl.BlockSpec((1, c), lambda i: (0, 0)),
            pl.BlockSpec((1, c), lambda i: (0, 0)),
        ],
        out_specs=[
            pl.BlockSpec((c, c), lambda i: (0, 0)),
            pl.BlockSpec((8, c), lambda i: (0, 0)),
        ],
        out_shape=[
            jax.ShapeDtypeStruct((c, c), jnp.float32),
            jax.ShapeDtypeStruct((8, c), jnp.float32),
        ],
    )(y, mom, g.reshape(1, -1), b.reshape(1, -1))


def _message_body(y_ref, mom_ref, g_ref, b_ref, xg_ref, w4_ref, c4_ref,
                  msg_ref):
    h = _bn_relu(y_ref, mom_ref, g_ref, b_ref)
    y4 = jnp.dot(h, w4_ref[...].T, preferred_element_type=jnp.float32) + c4_ref[...]
    x = xg_ref[:, 0:_D]
    acc = jnp.zeros((_EB, _D), jnp.float32)
    for d in range(_D):
        acc = acc + x[:, d:d + 1] * y4[:, d * _D:(d + 1) * _D]
    msg_ref[...] = acc


def _message(y, mom, g, b, xg, W4f, c4):
    e, c = y.shape
    cout = W4f.shape[0]
    return pl.pallas_call(
        _message_body,
        grid=(e // _EB,),
        in_specs=[
            pl.BlockSpec((_EB, c), lambda i: (i, 0)),
            pl.BlockSpec((8, c), lambda i: (0, 0)),
            pl.BlockSpec((1, c), lambda i: (0, 0)),
            pl.BlockSpec((1, c), lambda i: (0, 0)),
            pl.BlockSpec((_EB, 128), lambda i: (i, 0)),
            pl.BlockSpec((cout, c), lambda i: (0, 0)),
            pl.BlockSpec((1, cout), lambda i: (0, 0)),
        ],
        out_specs=pl.BlockSpec((_EB, _D), lambda i: (i, 0)),
        out_shape=jax.ShapeDtypeStruct((_EP, _D), jnp.float32),
    )(y, mom, g.reshape(1, -1), b.reshape(1, -1), xg, W4f, c4)


def _gru_body(p_ref, c_ref, h0_ref, bias_ref,
              wih_ref, whh_ref, bih_ref, bhh_ref, out_ref):
    cnt = jnp.maximum(c_ref[0][:, 0:1], 1.0)
    msg = jnp.maximum(p_ref[0] / cnt + bias_ref[...], 0.0)
    h0 = h0_ref[...]
    gi = (jnp.dot(msg, wih_ref[...].T, preferred_element_type=jnp.float32)
          + bih_ref[...])
    gh = jnp.dot(h0, whh_ref[...].T, preferred_element_type=jnp.float32) + bhh_ref[...]
    r = jax.nn.sigmoid(gi[:, 0:_D] + gh[:, 0:_D])
    z = jax.nn.sigmoid(gi[:, _D:2 * _D] + gh[:, _D:2 * _D])
    n = jnp.tanh(gi[:, 2 * _D:] + r * gh[:, 2 * _D:])
    out_ref[...] = (1.0 - z) * n + z * h0


def _gru(sums, counts, h0p, bias, W_ih, W_hh, b_ih, b_hh):
    return pl.pallas_call(
        _gru_body,
        grid=(2 * _NH // _NBH,),
        in_specs=[
            pl.BlockSpec((1, _NBH, _D), lambda i: (i // 16, i % 16, 0)),
            pl.BlockSpec((1, _NBH, 4), lambda i: (i // 16, i % 16, 0)),
            pl.BlockSpec((_NBH, _D), lambda i: (i, 0)),
            pl.BlockSpec((1, _D), lambda i: (0, 0)),
            pl.BlockSpec((3 * _D, _D), lambda i: (0, 0)),
            pl.BlockSpec((3 * _D, _D), lambda i: (0, 0)),
            pl.BlockSpec((1, 3 * _D), lambda i: (0, 0)),
            pl.BlockSpec((1, 3 * _D), lambda i: (0, 0)),
        ],
        out_specs=pl.BlockSpec((_NBH, _D), lambda i: (i, 0)),
        out_shape=jax.ShapeDtypeStruct((2 * _NH, _D), jnp.float32),
    )(sums, counts, h0p, bias.reshape(1, -1), W_ih, W_hh,
      b_ih.reshape(1, -1), b_hh.reshape(1, -1))


_NW = 32
_BPW = _EP // _NW
_NCH = _BPW // _CW
_SUB = 5
_RSB = _SUB * _CW


def _sc_gather(node, idx2d):
    @pl.kernel(out_type=jax.ShapeDtypeStruct((_EP, 128), jnp.float32),
               mesh=_sc_mesh(),
               scratch_types=[pltpu.VMEM((_NCH, _CW), jnp.int32),
                              pltpu.VMEM((_RSB, 128), jnp.float32),
                              pltpu.SemaphoreType.DMA])
    def k(node_hbm, idx_hbm, out_hbm, idx_v, rows_v, sem):
        wid = jax.lax.axis_index("subcore") * 2 + jax.lax.axis_index("core")
        base = pl.multiple_of(wid * _BPW, _RSB)
        pltpu.sync_copy(idx_hbm.at[wid], idx_v)

        @pl.loop(0, _NCH // _SUB)
        def _(sb):
            copies = [
                pltpu.async_copy(node_hbm.at[idx_v.at[sb * _SUB + j]],
                                 rows_v.at[pl.ds(j * _CW, _CW), :], sem)
                for j in range(_SUB)
            ]
            for cp in copies:
                cp.wait()
            off = pl.multiple_of(base + sb * _RSB, _RSB)
            pltpu.sync_copy(rows_v, out_hbm.at[pl.ds(off, _RSB), :])

    return k(node, idx2d)


def kernel(node, edge_index, edge, hidden, W1, g1, b1, W2, g2, b2, W3, g3, b3,
           W4, g4, b4, W_ih, W_hh, b_ih, b_hh, bias):
    pad = _EP - _E
    src = jnp.concatenate(
        [edge_index[:, 0], jnp.zeros((pad,), jnp.int32)]).reshape(_NW, _NCH, _CW)
    dstf = jnp.concatenate([edge_index[:, 1], jnp.full((pad,), _N, jnp.int32)])
    idx0 = jnp.where(dstf < _NH, dstf, _NAH - 1)
    idx1 = jnp.where(dstf >= _NH, dstf - _NH, _NAH - 1)
    idxs = jnp.stack([idx0.reshape(16, _NCH2, _CW),
                      idx1.reshape(16, _NCH2, _CW)])

    node_p = jnp.concatenate([node, jnp.zeros((_N, 128 - _D), jnp.float32)],
                             axis=1)
    xg = _sc_gather(node_p, src)

    y1, m1 = _lin_first(edge, W1)
    y2, m2 = _lin_mid(y1, m1, g1, b1, W2)
    y3, m3 = _lin_mid(y2, m2, g2, b2, W3)
    gram, hs = _gram(y3, m3, g3, b3)

    hmean = hs[0] / _E
    m4 = W4 @ hmean
    e2 = jnp.sum((W4 @ (gram / _E)) * W4, axis=1)
    var4 = e2 - m4 * m4
    a4 = g4 * jax.lax.rsqrt(var4 + _EPS)
    c4 = (b4 - m4 * a4).reshape(1, -1)
    W4f = W4 * a4[:, None]

    msg = _message(y3, m3, g3, b3, xg, W4f, c4)
    i0 = idxs[0].reshape(-1)
    i1 = idxs[1].reshape(-1)
    s0 = jax.ops.segment_sum(msg, i0, num_segments=_NAH)
    s1 = jax.ops.segment_sum(msg, i1, num_segments=_NAH)
    onesv = jnp.ones((_EP,), jnp.float32)
    c0 = jnp.broadcast_to(
        jax.ops.segment_sum(onesv, i0, num_segments=_NAH)[:, None], (_NAH, 4))
    c1 = jnp.broadcast_to(
        jax.ops.segment_sum(onesv, i1, num_segments=_NAH)[:, None], (_NAH, 4))
    sums = jnp.stack([s0, s1])
    counts = jnp.stack([c0, c1])

    h0p = jnp.concatenate(
        [hidden[0], jnp.zeros((2 * _NH - _N, _D), jnp.float32)], axis=0)
    h_new_p = _gru(sums, counts, h0p, bias, W_ih, W_hh, b_ih, b_hh)
    h_new = h_new_p[:_N]
    return h_new, h_new[None, :, :]


# speedup vs baseline: 1.2430x; 1.2082x over previous
import jax
import jax.numpy as jnp
from jax.experimental import pallas as pl
from jax.experimental.pallas import tpu as pltpu
from jax.experimental.pallas import tpu_sc as plsc

_N = 50000
_E = 100000
_EP = 102400
_D = 32
_EB = 2000
_NB = 2000
_CW = 128
_NH = 25088
_NAH = 25600
_SRH = 1600
_NCH2 = 50
_NBH = 1568
_EPS = 1e-5


def _sc_mesh():
    return plsc.VectorSubcoreMesh(core_axis_name="core", subcore_axis_name="subcore")


def _lin_first_body(x_ref, w_ref, y_ref, mom_ref):
    i = pl.program_id(0)
    y = jnp.dot(x_ref[...], w_ref[...].T, preferred_element_type=jnp.float32)
    y_ref[...] = y

    @pl.when(i == 0)
    def _():
        mom_ref[...] = jnp.zeros_like(mom_ref)

    mom_ref[0:1, :] += jnp.sum(y, axis=0, keepdims=True)
    mom_ref[1:2, :] += jnp.sum(y * y, axis=0, keepdims=True)


def _lin_first(x, W):
    e, cin = x.shape
    cout = W.shape[0]
    return pl.pallas_call(
        _lin_first_body,
        grid=(e // _EB,),
        in_specs=[
            pl.BlockSpec((_EB, cin), lambda i: (i, 0)),
            pl.BlockSpec((cout, cin), lambda i: (0, 0)),
        ],
        out_specs=[
            pl.BlockSpec((_EB, cout), lambda i: (i, 0)),
            pl.BlockSpec((8, cout), lambda i: (0, 0)),
        ],
        out_shape=[
            jax.ShapeDtypeStruct((e, cout), jnp.float32),
            jax.ShapeDtypeStruct((8, cout), jnp.float32),
        ],
    )(x, W)


def _bn_relu(y_ref, mom_ref, g_ref, b_ref):
    inv_e = 1.0 / _E
    mean = mom_ref[0:1, :] * inv_e
    var = mom_ref[1:2, :] * inv_e - mean * mean
    a = g_ref[...] * jax.lax.rsqrt(var + _EPS)
    c = b_ref[...] - mean * a
    return jnp.maximum(y_ref[...] * a + c, 0.0)


def _lin_mid_body(y_ref, mom_ref, g_ref, b_ref, w_ref, yo_ref, mo_ref):
    i = pl.program_id(0)
    h = _bn_relu(y_ref, mom_ref, g_ref, b_ref)
    y = jnp.dot(h, w_ref[...].T, preferred_element_type=jnp.float32)
    yo_ref[...] = y

    @pl.when(i == 0)
    def _():
        mo_ref[...] = jnp.zeros_like(mo_ref)

    mo_ref[0:1, :] += jnp.sum(y, axis=0, keepdims=True)
    mo_ref[1:2, :] += jnp.sum(y * y, axis=0, keepdims=True)


def _lin_mid(y, mom, g, b, W):
    e, cin = y.shape
    cout = W.shape[0]
    return pl.pallas_call(
        _lin_mid_body,
        grid=(e // _EB,),
        in_specs=[
            pl.BlockSpec((_EB, cin), lambda i: (i, 0)),
            pl.BlockSpec((8, cin), lambda i: (0, 0)),
            pl.BlockSpec((1, cin), lambda i: (0, 0)),
            pl.BlockSpec((1, cin), lambda i: (0, 0)),
            pl.BlockSpec((cout, cin), lambda i: (0, 0)),
        ],
        out_specs=[
            pl.BlockSpec((_EB, cout), lambda i: (i, 0)),
            pl.BlockSpec((8, cout), lambda i: (0, 0)),
        ],
        out_shape=[
            jax.ShapeDtypeStruct((e, cout), jnp.float32),
            jax.ShapeDtypeStruct((8, cout), jnp.float32),
        ],
    )(y, mom, g.reshape(1, -1), b.reshape(1, -1), W)


def _gram_body(y_ref, mom_ref, g_ref, b_ref, gram_ref, hs_ref):
    i = pl.program_id(0)
    h = _bn_relu(y_ref, mom_ref, g_ref, b_ref)

    @pl.when(i == 0)
    def _():
        gram_ref[...] = jnp.zeros_like(gram_ref)
        hs_ref[...] = jnp.zeros_like(hs_ref)

    gram_ref[...] += jax.lax.dot_general(
        h, h, (((0,), (0,)), ((), ())), preferred_element_type=jnp.float32)
    hs_ref[0:1, :] += jnp.sum(h, axis=0, keepdims=True)


def _gram(y, mom, g, b):
    e, c = y.shape
    return pl.pallas_call(
        _gram_body,
        grid=(e // _EB,),
        in_specs=[
            pl.BlockSpec((_EB, c), lambda i: (i, 0)),
            pl.BlockSpec((8, c), lambda i: (0, 0)),
            pl.BlockSpec((1, c), lambda i: (0, 0)),
            pl.BlockSpec((1, c), lambda i: (0, 0)),
        ],
        out_specs=[
            pl.BlockSpec((c, c), lambda i: (0, 0)),
            pl.BlockSpec((8, c), lambda i: (0, 0)),
        ],
        out_shape=[
            jax.ShapeDtypeStruct((c, c), jnp.float32),
            jax.ShapeDtypeStruct((8, c), jnp.float32),
        ],
    )(y, mom, g.reshape(1, -1), b.reshape(1, -1))


def _message_body(y_ref, mom_ref, g_ref, b_ref, xg_ref, w4_ref, c4_ref,
                  msg_ref):
    h = _bn_relu(y_ref, mom_ref, g_ref, b_ref)
    y4 = jnp.dot(h, w4_ref[...].T, preferred_element_type=jnp.float32) + c4_ref[...]
    x = xg_ref[:, 0:_D]
    acc = jnp.zeros((_EB, _D), jnp.float32)
    for d in range(_D):
        acc = acc + x[:, d:d + 1] * y4[:, d * _D:(d + 1) * _D]
    msg_ref[:, 0:_D] = acc
    msg_ref[:, _D:40] = jnp.ones((_EB, 8), jnp.float32)


def _message(y, mom, g, b, xg, W4f, c4):
    e, c = y.shape
    cout = W4f.shape[0]
    return pl.pallas_call(
        _message_body,
        grid=(e // _EB,),
        in_specs=[
            pl.BlockSpec((_EB, c), lambda i: (i, 0)),
            pl.BlockSpec((8, c), lambda i: (0, 0)),
            pl.BlockSpec((1, c), lambda i: (0, 0)),
            pl.BlockSpec((1, c), lambda i: (0, 0)),
            pl.BlockSpec((_EB, 128), lambda i: (i, 0)),
            pl.BlockSpec((cout, c), lambda i: (0, 0)),
            pl.BlockSpec((1, cout), lambda i: (0, 0)),
        ],
        out_specs=pl.BlockSpec((_EB, 40), lambda i: (i, 0)),
        out_shape=jax.ShapeDtypeStruct((_EP, 40), jnp.float32),
    )(y, mom, g.reshape(1, -1), b.reshape(1, -1), xg, W4f, c4)


def _gru_body(s_ref, h0_ref, bias_ref, wih_ref, whh_ref, bih_ref, bhh_ref,
              out_ref):
    sv = s_ref[...]
    cnt = jnp.maximum(sv[:, _D:_D + 1], 1.0)
    msg = jnp.maximum(sv[:, 0:_D] / cnt + bias_ref[...], 0.0)
    h0 = h0_ref[...]
    gi = (jnp.dot(msg, wih_ref[...].T, preferred_element_type=jnp.float32)
          + bih_ref[...])
    gh = jnp.dot(h0, whh_ref[...].T, preferred_element_type=jnp.float32) + bhh_ref[...]
    r = jax.nn.sigmoid(gi[:, 0:_D] + gh[:, 0:_D])
    z = jax.nn.sigmoid(gi[:, _D:2 * _D] + gh[:, _D:2 * _D])
    n = jnp.tanh(gi[:, 2 * _D:] + r * gh[:, 2 * _D:])
    out_ref[...] = (1.0 - z) * n + z * h0


def _gru(sums, h0, bias, W_ih, W_hh, b_ih, b_hh):
    return pl.pallas_call(
        _gru_body,
        grid=(_N // _NB,),
        in_specs=[
            pl.BlockSpec((_NB, 40), lambda i: (i, 0)),
            pl.BlockSpec((_NB, _D), lambda i: (i, 0)),
            pl.BlockSpec((1, _D), lambda i: (0, 0)),
            pl.BlockSpec((3 * _D, _D), lambda i: (0, 0)),
            pl.BlockSpec((3 * _D, _D), lambda i: (0, 0)),
            pl.BlockSpec((1, 3 * _D), lambda i: (0, 0)),
            pl.BlockSpec((1, 3 * _D), lambda i: (0, 0)),
        ],
        out_specs=pl.BlockSpec((_NB, _D), lambda i: (i, 0)),
        out_shape=jax.ShapeDtypeStruct((_N, _D), jnp.float32),
    )(sums, h0, bias.reshape(1, -1), W_ih, W_hh,
      b_ih.reshape(1, -1), b_hh.reshape(1, -1))


_NW = 32
_BPW = _EP // _NW
_NCH = _BPW // _CW
_SUB = 5
_RSB = _SUB * _CW


def _sc_gather(node, idx2d):
    @pl.kernel(out_type=jax.ShapeDtypeStruct((_EP, 128), jnp.float32),
               mesh=_sc_mesh(),
               scratch_types=[pltpu.VMEM((_NCH, _CW), jnp.int32),
                              pltpu.VMEM((_RSB, 128), jnp.float32),
                              pltpu.SemaphoreType.DMA])
    def k(node_hbm, idx_hbm, out_hbm, idx_v, rows_v, sem):
        wid = jax.lax.axis_index("subcore") * 2 + jax.lax.axis_index("core")
        base = pl.multiple_of(wid * _BPW, _RSB)
        pltpu.sync_copy(idx_hbm.at[wid], idx_v)

        @pl.loop(0, _NCH // _SUB)
        def _(sb):
            copies = [
                pltpu.async_copy(node_hbm.at[idx_v.at[sb * _SUB + j]],
                                 rows_v.at[pl.ds(j * _CW, _CW), :], sem)
                for j in range(_SUB)
            ]
            for cp in copies:
                cp.wait()
            off = pl.multiple_of(base + sb * _RSB, _RSB)
            pltpu.sync_copy(rows_v, out_hbm.at[pl.ds(off, _RSB), :])

    return k(node, idx2d)


def kernel(node, edge_index, edge, hidden, W1, g1, b1, W2, g2, b2, W3, g3, b3,
           W4, g4, b4, W_ih, W_hh, b_ih, b_hh, bias):
    pad = _EP - _E
    src = jnp.concatenate(
        [edge_index[:, 0], jnp.zeros((pad,), jnp.int32)]).reshape(_NW, _NCH, _CW)
    dstf = jnp.concatenate([edge_index[:, 1], jnp.full((pad,), _N, jnp.int32)])

    node_p = jnp.concatenate([node, jnp.zeros((_N, 128 - _D), jnp.float32)],
                             axis=1)
    xg = _sc_gather(node_p, src)

    y1, m1 = _lin_first(edge, W1)
    y2, m2 = _lin_mid(y1, m1, g1, b1, W2)
    y3, m3 = _lin_mid(y2, m2, g2, b2, W3)
    gram, hs = _gram(y3, m3, g3, b3)

    hmean = hs[0] / _E
    m4 = W4 @ hmean
    e2 = jnp.sum((W4 @ (gram / _E)) * W4, axis=1)
    var4 = e2 - m4 * m4
    a4 = g4 * jax.lax.rsqrt(var4 + _EPS)
    c4 = (b4 - m4 * a4).reshape(1, -1)
    W4f = W4 * a4[:, None]

    msgc = _message(y3, m3, g3, b3, xg, W4f, c4)
    sums = jax.ops.segment_sum(msgc, dstf, num_segments=_N + 8)

    h_new = _gru(sums, hidden[0], bias, W_ih, W_hh, b_ih, b_hh)
    return h_new, h_new[None, :, :]


# speedup vs baseline: 1.2433x; 1.0002x over previous
import jax
import jax.numpy as jnp
from jax.experimental import pallas as pl
from jax.experimental.pallas import tpu as pltpu
from jax.experimental.pallas import tpu_sc as plsc

_N = 50000
_E = 100000
_EP = 102400
_D = 32
_EB = 2000
_NB = 2000
_CW = 128
_NH = 25088
_NAH = 25600
_SRH = 1600
_NCH2 = 50
_NBH = 1568
_EPS = 1e-5


def _sc_mesh():
    return plsc.VectorSubcoreMesh(core_axis_name="core", subcore_axis_name="subcore")


def _lin_first_body(x_ref, w_ref, y_ref, mom_ref):
    i = pl.program_id(0)
    y = jnp.dot(x_ref[...], w_ref[...].T, preferred_element_type=jnp.float32)
    y_ref[...] = y

    @pl.when(i == 0)
    def _():
        mom_ref[...] = jnp.zeros_like(mom_ref)

    mom_ref[0:1, :] += jnp.sum(y, axis=0, keepdims=True)
    mom_ref[1:2, :] += jnp.sum(y * y, axis=0, keepdims=True)


def _lin_first(x, W):
    e, cin = x.shape
    cout = W.shape[0]
    return pl.pallas_call(
        _lin_first_body,
        grid=(e // _EB,),
        in_specs=[
            pl.BlockSpec((_EB, cin), lambda i: (i, 0)),
            pl.BlockSpec((cout, cin), lambda i: (0, 0)),
        ],
        out_specs=[
            pl.BlockSpec((_EB, cout), lambda i: (i, 0)),
            pl.BlockSpec((8, cout), lambda i: (0, 0)),
        ],
        out_shape=[
            jax.ShapeDtypeStruct((e, cout), jnp.float32),
            jax.ShapeDtypeStruct((8, cout), jnp.float32),
        ],
    )(x, W)


def _bn_relu(y_ref, mom_ref, g_ref, b_ref):
    inv_e = 1.0 / _E
    mean = mom_ref[0:1, :] * inv_e
    var = mom_ref[1:2, :] * inv_e - mean * mean
    a = g_ref[...] * jax.lax.rsqrt(var + _EPS)
    c = b_ref[...] - mean * a
    return jnp.maximum(y_ref[...] * a + c, 0.0)


def _lin_mid_body(y_ref, mom_ref, g_ref, b_ref, w_ref, yo_ref, mo_ref):
    i = pl.program_id(0)
    h = _bn_relu(y_ref, mom_ref, g_ref, b_ref)
    y = jnp.dot(h, w_ref[...].T, preferred_element_type=jnp.float32)
    yo_ref[...] = y

    @pl.when(i == 0)
    def _():
        mo_ref[...] = jnp.zeros_like(mo_ref)

    mo_ref[0:1, :] += jnp.sum(y, axis=0, keepdims=True)
    mo_ref[1:2, :] += jnp.sum(y * y, axis=0, keepdims=True)


def _lin_mid(y, mom, g, b, W):
    e, cin = y.shape
    cout = W.shape[0]
    return pl.pallas_call(
        _lin_mid_body,
        grid=(e // _EB,),
        in_specs=[
            pl.BlockSpec((_EB, cin), lambda i: (i, 0)),
            pl.BlockSpec((8, cin), lambda i: (0, 0)),
            pl.BlockSpec((1, cin), lambda i: (0, 0)),
            pl.BlockSpec((1, cin), lambda i: (0, 0)),
            pl.BlockSpec((cout, cin), lambda i: (0, 0)),
        ],
        out_specs=[
            pl.BlockSpec((_EB, cout), lambda i: (i, 0)),
            pl.BlockSpec((8, cout), lambda i: (0, 0)),
        ],
        out_shape=[
            jax.ShapeDtypeStruct((e, cout), jnp.float32),
            jax.ShapeDtypeStruct((8, cout), jnp.float32),
        ],
    )(y, mom, g.reshape(1, -1), b.reshape(1, -1), W)


def _gram_body(y_ref, mom_ref, g_ref, b_ref, gram_ref, hs_ref):
    i = pl.program_id(0)
    h = _bn_relu(y_ref, mom_ref, g_ref, b_ref)

    @pl.when(i == 0)
    def _():
        gram_ref[...] = jnp.zeros_like(gram_ref)
        hs_ref[...] = jnp.zeros_like(hs_ref)

    gram_ref[...] += jax.lax.dot_general(
        h, h, (((0,), (0,)), ((), ())), preferred_element_type=jnp.float32)
    hs_ref[0:1, :] += jnp.sum(h, axis=0, keepdims=True)


def _gram(y, mom, g, b):
    e, c = y.shape
    return pl.pallas_call(
        _gram_body,
        grid=(e // _EB,),
        in_specs=[
            pl.BlockSpec((_EB, c), lambda i: (i, 0)),
            pl.BlockSpec((8, c), lambda i: (0, 0)),
            pl.BlockSpec((1, c), lambda i: (0, 0)),
            pl.BlockSpec((1, c), lambda i: (0, 0)),
        ],
        out_specs=[
            pl.BlockSpec((c, c), lambda i: (0, 0)),
            pl.BlockSpec((8, c), lambda i: (0, 0)),
        ],
        out_shape=[
            jax.ShapeDtypeStruct((c, c), jnp.float32),
            jax.ShapeDtypeStruct((8, c), jnp.float32),
        ],
    )(y, mom, g.reshape(1, -1), b.reshape(1, -1))


def _message_body(y_ref, mom_ref, g_ref, b_ref, xg_ref, w4_ref, c4_ref,
                  msg_ref):
    h = _bn_relu(y_ref, mom_ref, g_ref, b_ref)
    y4 = jnp.dot(h, w4_ref[...].T, preferred_element_type=jnp.float32) + c4_ref[...]
    x = xg_ref[:, 0:_D]
    acc = jnp.zeros((_EB, _D), jnp.float32)
    for d in range(_D):
        acc = acc + x[:, d:d + 1] * y4[:, d * _D:(d + 1) * _D]
    msg_ref[:, 0:_D] = acc
    msg_ref[:, _D:40] = jnp.ones((_EB, 8), jnp.float32)


def _message(y, mom, g, b, xg, W4f, c4):
    e, c = y.shape
    cout = W4f.shape[0]
    return pl.pallas_call(
        _message_body,
        grid=(e // _EB,),
        in_specs=[
            pl.BlockSpec((_EB, c), lambda i: (i, 0)),
            pl.BlockSpec((8, c), lambda i: (0, 0)),
            pl.BlockSpec((1, c), lambda i: (0, 0)),
            pl.BlockSpec((1, c), lambda i: (0, 0)),
            pl.BlockSpec((_EB, 128), lambda i: (i, 0)),
            pl.BlockSpec((cout, c), lambda i: (0, 0)),
            pl.BlockSpec((1, cout), lambda i: (0, 0)),
        ],
        out_specs=pl.BlockSpec((_EB, 40), lambda i: (i, 0)),
        out_shape=jax.ShapeDtypeStruct((_EP, 40), jnp.float32),
    )(y, mom, g.reshape(1, -1), b.reshape(1, -1), xg, W4f, c4)


def _gru_body(s_ref, h0_ref, bias_ref, wih_ref, whh_ref, bih_ref, bhh_ref,
              out_ref):
    sv = s_ref[...]
    cnt = jnp.maximum(sv[:, _D:_D + 1], 1.0)
    msg = jnp.maximum(sv[:, 0:_D] / cnt + bias_ref[...], 0.0)
    h0 = h0_ref[...]
    gi = (jnp.dot(msg, wih_ref[...].T, preferred_element_type=jnp.float32)
          + bih_ref[...])
    gh = jnp.dot(h0, whh_ref[...].T, preferred_element_type=jnp.float32) + bhh_ref[...]
    r = jax.nn.sigmoid(gi[:, 0:_D] + gh[:, 0:_D])
    z = jax.nn.sigmoid(gi[:, _D:2 * _D] + gh[:, _D:2 * _D])
    n = jnp.tanh(gi[:, 2 * _D:] + r * gh[:, 2 * _D:])
    out_ref[...] = (1.0 - z) * n + z * h0


def _gru(sums, h0, bias, W_ih, W_hh, b_ih, b_hh):
    return pl.pallas_call(
        _gru_body,
        grid=(_N // _NB,),
        in_specs=[
            pl.BlockSpec((_NB, 40), lambda i: (i, 0)),
            pl.BlockSpec((_NB, _D), lambda i: (i, 0)),
            pl.BlockSpec((1, _D), lambda i: (0, 0)),
            pl.BlockSpec((3 * _D, _D), lambda i: (0, 0)),
            pl.BlockSpec((3 * _D, _D), lambda i: (0, 0)),
            pl.BlockSpec((1, 3 * _D), lambda i: (0, 0)),
            pl.BlockSpec((1, 3 * _D), lambda i: (0, 0)),
        ],
        out_specs=pl.BlockSpec((_NB, _D), lambda i: (i, 0)),
        out_shape=jax.ShapeDtypeStruct((_N, _D), jnp.float32),
    )(sums, h0, bias.reshape(1, -1), W_ih, W_hh,
      b_ih.reshape(1, -1), b_hh.reshape(1, -1))


_NW = 32
_BPW = _EP // _NW
_NCH = _BPW // _CW
_SUB = 5
_RSB = _SUB * _CW


def _sc_gather(node_p, idx2d):
    @pl.kernel(out_type=jax.ShapeDtypeStruct((_EP, 128), jnp.float32),
               mesh=_sc_mesh(),
               scratch_types=[pltpu.VMEM((_NCH, _CW), jnp.int32),
                              pltpu.VMEM((_CW, 128), jnp.float32),
                              pltpu.VMEM((_CW, 128), jnp.float32),
                              pltpu.SemaphoreType.DMA,
                              pltpu.SemaphoreType.DMA])
    def k(node_hbm, idx_hbm, out_hbm, idx_v, buf0, buf1, sem0, sem1):
        wid = jax.lax.axis_index("subcore") * 2 + jax.lax.axis_index("core")
        base = pl.multiple_of(wid * _BPW, _CW)
        pltpu.sync_copy(idx_hbm.at[wid], idx_v)

        bufs = (buf0, buf1)
        sems = (sem0, sem1)
        pltpu.async_copy(node_hbm.at[idx_v.at[0]], buf0, sem0).wait()

        @pl.loop(0, _NCH - 1)
        def _(j):
            @pl.when(j % 2 == 0)
            def _():
                cp = pltpu.async_copy(node_hbm.at[idx_v.at[j + 1]], buf1, sem1)
                off = pl.multiple_of(base + j * _CW, _CW)
                pltpu.sync_copy(buf0, out_hbm.at[pl.ds(off, _CW), :])
                cp.wait()

            @pl.when(j % 2 == 1)
            def _():
                cp = pltpu.async_copy(node_hbm.at[idx_v.at[j + 1]], buf0, sem0)
                off = pl.multiple_of(base + j * _CW, _CW)
                pltpu.sync_copy(buf1, out_hbm.at[pl.ds(off, _CW), :])
                cp.wait()

        off = pl.multiple_of(base + (_NCH - 1) * _CW, _CW)
        pltpu.sync_copy(buf0, out_hbm.at[pl.ds(off, _CW), :])

    return k(node_p, idx2d)


def kernel(node, edge_index, edge, hidden, W1, g1, b1, W2, g2, b2, W3, g3, b3,
           W4, g4, b4, W_ih, W_hh, b_ih, b_hh, bias):
    pad = _EP - _E
    src = jnp.concatenate(
        [edge_index[:, 0], jnp.zeros((pad,), jnp.int32)]).reshape(_NW, _NCH, _CW)
    dstf = jnp.concatenate([edge_index[:, 1], jnp.full((pad,), _N, jnp.int32)])

    node_p = jnp.concatenate([node, jnp.zeros((_N, 128 - _D), jnp.float32)],
                             axis=1)
    xg = _sc_gather(node_p, src)

    y1, m1 = _lin_first(edge, W1)
    y2, m2 = _lin_mid(y1, m1, g1, b1, W2)
    y3, m3 = _lin_mid(y2, m2, g2, b2, W3)
    gram, hs = _gram(y3, m3, g3, b3)

    hmean = hs[0] / _E
    m4 = W4 @ hmean
    e2 = jnp.sum((W4 @ (gram / _E)) * W4, axis=1)
    var4 = e2 - m4 * m4
    a4 = g4 * jax.lax.rsqrt(var4 + _EPS)
    c4 = (b4 - m4 * a4).reshape(1, -1)
    W4f = W4 * a4[:, None]

    msgc = _message(y3, m3, g3, b3, xg, W4f, c4)
    sums = jax.ops.segment_sum(msgc, dstf, num_segments=_N + 8)

    h_new = _gru(sums, hidden[0], bias, W_ih, W_hh, b_ih, b_hh)
    return h_new, h_new[None, :, :]
